# baseline (speedup 1.0000x reference)
import functools

import jax
import jax.numpy as jnp
from jax.experimental import pallas as pl
from jax.experimental.pallas import tpu as pltpu

Q = 1024
D = 16
K_TOTAL = 100000
BK = 6144
NSTEPS = (K_TOTAL + BK - 1) // BK


def _nn_kernel(q_ref, kt_ref, score_ref, idx_ref):
    step = pl.program_id(0)

    q = q_ref[...]
    kt = kt_ref[...]

    qm2 = -2.0 * q
    qk2 = jax.lax.dot_general(
        qm2, kt, (((1,), (0,)), ((), ())),
        preferred_element_type=jnp.float32)
    q_sq = jnp.sum(q * q, axis=1, keepdims=True)
    k_sq = jnp.sum(kt * kt, axis=0, keepdims=True)
    dists = q_sq + qk2 + k_sq

    lcol = jax.lax.broadcasted_iota(
        jnp.int32, (1, BK), 1).astype(jnp.float32)

    def _minmin(d):
        bmin = jnp.min(d, axis=1, keepdims=True)
        bidx_f = jnp.min(jnp.where(d == bmin, lcol, float(BK)),
                         axis=1, keepdims=True)
        bidx = bidx_f.astype(jnp.int32) + step * BK
        return bmin, bidx

    @pl.when(step == 0)
    def _init():
        bmin, bidx = _minmin(dists)
        score_ref[...] = bmin
        idx_ref[...] = bidx

    @pl.when(jnp.logical_and(step > 0, step < NSTEPS - 1))
    def _update():
        bmin, bidx = _minmin(dists)
        run = score_ref[...]
        better = bmin < run
        score_ref[...] = jnp.where(better, bmin, run)
        idx_ref[...] = jnp.where(better, bidx, idx_ref[...])

    @pl.when(step == NSTEPS - 1)
    def _tail():
        masked = jnp.where(lcol < K_TOTAL - step * BK, dists, jnp.inf)
        bmin, bidx = _minmin(masked)
        run = score_ref[...]
        better = bmin < run
        score_ref[...] = jnp.sqrt(jnp.maximum(
            jnp.where(better, bmin, run), 0.0))
        idx_ref[...] = jnp.where(better, bidx, idx_ref[...])


@functools.partial(jax.jit, static_argnames=())
def kernel(queries, keys):
    keys_t = keys.T
    scores, nn_idx = pl.pallas_call(
        _nn_kernel,
        grid=(NSTEPS,),
        in_specs=[
            pl.BlockSpec((Q, D), lambda i: (0, 0)),
            pl.BlockSpec((D, BK), lambda i: (0, i)),
        ],
        out_specs=[
            pl.BlockSpec((Q, 1), lambda i: (0, 0)),
            pl.BlockSpec((Q, 1), lambda i: (0, 0)),
        ],
        out_shape=[
            jax.ShapeDtypeStruct((Q, 1), jnp.float32),
            jax.ShapeDtypeStruct((Q, 1), jnp.int32),
        ],
        compiler_params=pltpu.CompilerParams(
            dimension_semantics=("arbitrary",),
        ),
    )(queries, keys_t)
    return scores[:, 0], nn_idx


# speedup vs baseline: 1.0153x; 1.0153x over previous
import functools

import jax
import jax.numpy as jnp
from jax.experimental import pallas as pl
from jax.experimental.pallas import tpu as pltpu

Q = 1024
D = 16
K_TOTAL = 100000
BK = 5120
NSTEPS = (K_TOTAL + BK - 1) // BK


def _nn_kernel(q_ref, kt_ref, score_ref, idx_ref):
    step = pl.program_id(0)

    q = q_ref[...]
    kt = kt_ref[...]

    qm2 = -2.0 * q
    qk2 = jax.lax.dot_general(
        qm2, kt, (((1,), (0,)), ((), ())),
        preferred_element_type=jnp.float32)
    q_sq = jnp.sum(q * q, axis=1, keepdims=True)
    k_sq = jnp.sum(kt * kt, axis=0, keepdims=True)
    dists = q_sq + qk2 + k_sq

    lcol = jax.lax.broadcasted_iota(
        jnp.int32, (1, BK), 1).astype(jnp.float32)

    def _minmin(d):
        bmin = jnp.min(d, axis=1, keepdims=True)
        bidx_f = jnp.min(jnp.where(d == bmin, lcol, float(BK)),
                         axis=1, keepdims=True)
        bidx = bidx_f.astype(jnp.int32) + step * BK
        return bmin, bidx

    @pl.when(step == 0)
    def _init():
        bmin, bidx = _minmin(dists)
        score_ref[...] = bmin
        idx_ref[...] = bidx

    @pl.when(jnp.logical_and(step > 0, step < NSTEPS - 1))
    def _update():
        bmin, bidx = _minmin(dists)
        run = score_ref[...]
        better = bmin < run
        score_ref[...] = jnp.where(better, bmin, run)
        idx_ref[...] = jnp.where(better, bidx, idx_ref[...])

    @pl.when(step == NSTEPS - 1)
    def _tail():
        masked = jnp.where(lcol < K_TOTAL - step * BK, dists, jnp.inf)
        bmin, bidx = _minmin(masked)
        run = score_ref[...]
        better = bmin < run
        score_ref[...] = jnp.sqrt(jnp.maximum(
            jnp.where(better, bmin, run), 0.0))
        idx_ref[...] = jnp.where(better, bidx, idx_ref[...])


@functools.partial(jax.jit, static_argnames=())
def kernel(queries, keys):
    keys_t = keys.T
    scores, nn_idx = pl.pallas_call(
        _nn_kernel,
        grid=(NSTEPS,),
        in_specs=[
            pl.BlockSpec((Q, D), lambda i: (0, 0)),
            pl.BlockSpec((D, BK), lambda i: (0, i)),
        ],
        out_specs=[
            pl.BlockSpec((Q, 1), lambda i: (0, 0)),
            pl.BlockSpec((Q, 1), lambda i: (0, 0)),
        ],
        out_shape=[
            jax.ShapeDtypeStruct((Q, 1), jnp.float32),
            jax.ShapeDtypeStruct((Q, 1), jnp.int32),
        ],
        compiler_params=pltpu.CompilerParams(
            dimension_semantics=("arbitrary",),
        ),
    )(queries, keys_t)
    return scores[:, 0], nn_idx


# speedup vs baseline: 1.0399x; 1.0243x over previous
import functools

import jax
import jax.numpy as jnp
from jax.experimental import pallas as pl
from jax.experimental.pallas import tpu as pltpu

Q = 1024
D = 16
K_TOTAL = 100000
BK = 5888
NSTEPS = (K_TOTAL + BK - 1) // BK


def _nn_kernel(q_ref, kt_ref, score_ref, idx_ref):
    step = pl.program_id(0)

    q = q_ref[...]
    kt = kt_ref[...]

    qm2 = -2.0 * q
    qk2 = jax.lax.dot_general(
        qm2, kt, (((1,), (0,)), ((), ())),
        preferred_element_type=jnp.float32)
    q_sq = jnp.sum(q * q, axis=1, keepdims=True)
    k_sq = jnp.sum(kt * kt, axis=0, keepdims=True)
    dists = q_sq + qk2 + k_sq

    lcol = jax.lax.broadcasted_iota(
        jnp.int32, (1, BK), 1).astype(jnp.float32)

    def _minmin(d):
        bmin = jnp.min(d, axis=1, keepdims=True)
        bidx_f = jnp.min(jnp.where(d == bmin, lcol, float(BK)),
                         axis=1, keepdims=True)
        bidx = bidx_f.astype(jnp.int32) + step * BK
        return bmin, bidx

    @pl.when(step == 0)
    def _init():
        bmin, bidx = _minmin(dists)
        score_ref[...] = bmin
        idx_ref[...] = bidx

    @pl.when(jnp.logical_and(step > 0, step < NSTEPS - 1))
    def _update():
        bmin, bidx = _minmin(dists)
        run = score_ref[...]
        better = bmin < run
        score_ref[...] = jnp.where(better, bmin, run)
        idx_ref[...] = jnp.where(better, bidx, idx_ref[...])

    @pl.when(step == NSTEPS - 1)
    def _tail():
        masked = jnp.where(lcol < K_TOTAL - step * BK, dists, jnp.inf)
        bmin, bidx = _minmin(masked)
        run = score_ref[...]
        better = bmin < run
        score_ref[...] = jnp.sqrt(jnp.maximum(
            jnp.where(better, bmin, run), 0.0))
        idx_ref[...] = jnp.where(better, bidx, idx_ref[...])


@functools.partial(jax.jit, static_argnames=())
def kernel(queries, keys):
    keys_t = keys.T
    scores, nn_idx = pl.pallas_call(
        _nn_kernel,
        grid=(NSTEPS,),
        in_specs=[
            pl.BlockSpec((Q, D), lambda i: (0, 0)),
            pl.BlockSpec((D, BK), lambda i: (0, i)),
        ],
        out_specs=[
            pl.BlockSpec((Q, 1), lambda i: (0, 0)),
            pl.BlockSpec((Q, 1), lambda i: (0, 0)),
        ],
        out_shape=[
            jax.ShapeDtypeStruct((Q, 1), jnp.float32),
            jax.ShapeDtypeStruct((Q, 1), jnp.int32),
        ],
        compiler_params=pltpu.CompilerParams(
            dimension_semantics=("arbitrary",),
        ),
    )(queries, keys_t)
    return scores[:, 0], nn_idx
